# Initial kernel scaffold; baseline (speedup 1.0000x reference)
import jax
import jax.numpy as jnp
from jax.experimental import pallas as pl


def kernel(x, neuron_emb, W_proj, W_neuron, circuit_emb, top_k):
    raise NotImplementedError("write your pallas kernel here")


# speedup vs baseline: 1.4421x; 1.4421x over previous
import functools

import jax
import jax.numpy as jnp
from jax import lax
from jax.experimental import pallas as pl
from jax.experimental.pallas import tpu as pltpu

B = 4096
D_MODEL = 2048
D_SPACE = 64
N_R = 64
NEURONS = 64
BT = 512


def _softmax_last(v):
    m = jnp.max(v, axis=-1, keepdims=True)
    e = jnp.exp(v - m)
    return e / jnp.sum(e, axis=-1, keepdims=True)


def _body(x_ref, wcat_ref, ce_ref, ne_ref, out_ref, en_ref, nn_ref):
    i = pl.program_id(0)

    @pl.when(i == 0)
    def _init():
        ce = ce_ref[...]
        en_ref[...] = ce / (jnp.sqrt(jnp.sum(ce * ce, axis=-1, keepdims=True)) + 1e-12)
        ne = ne_ref[...]
        nn_ref[...] = ne / (jnp.sqrt(jnp.sum(ne * ne, axis=-1, keepdims=True)) + 1e-12)

    x = x_ref[...]
    hcat = lax.dot_general(x, wcat_ref[...], (((1,), (1,)), ((), ())),
                           preferred_element_type=jnp.float32)
    h = hcat[:, :D_SPACE]
    hn = hcat[:, D_SPACE:]

    logits = lax.dot_general(h, en_ref[...], (((1,), (1,)), ((), ())),
                             preferred_element_type=jnp.float32)

    iota = lax.broadcasted_iota(jnp.int32, logits.shape, 1)
    v1 = jnp.max(logits, axis=-1, keepdims=True)
    i1 = jnp.min(jnp.where(logits == v1, iota, N_R), axis=-1, keepdims=True)
    one1 = iota == i1
    masked = jnp.where(one1, -jnp.inf, logits)
    v2 = jnp.max(masked, axis=-1, keepdims=True)
    i2 = jnp.min(jnp.where(masked == v2, iota, N_R), axis=-1, keepdims=True)
    one2 = iota == i2

    e = jnp.exp(v2 - v1)
    g1 = 1.0 / (1.0 + e)
    g2 = e / (1.0 + e)

    full = lax.dot_general(hn, nn_ref[...], (((1,), (1,)), ((), ())),
                           preferred_element_type=jnp.float32)
    full3 = full.reshape(BT, NEURONS, N_R)

    nl1 = jnp.sum(jnp.where(one1[:, None, :], full3, 0.0), axis=-1)
    nl2 = jnp.sum(jnp.where(one2[:, None, :], full3, 0.0), axis=-1)

    out_ref[:, 0, :] = _softmax_last(nl1) * g1
    out_ref[:, 1, :] = _softmax_last(nl2) * g2


@functools.partial(jax.jit, static_argnames=())
def _run(x, wcat, ce_r, ne_t):
    grid = (B // BT,)
    return pl.pallas_call(
        _body,
        grid=grid,
        in_specs=[
            pl.BlockSpec((BT, D_MODEL), lambda i: (i, 0)),
            pl.BlockSpec((2 * D_SPACE, D_MODEL), lambda i: (0, 0)),
            pl.BlockSpec((N_R, D_SPACE), lambda i: (0, 0)),
            pl.BlockSpec((NEURONS * N_R, D_SPACE), lambda i: (0, 0)),
        ],
        out_specs=pl.BlockSpec((BT, 2, NEURONS), lambda i: (i, 0, 0)),
        out_shape=jax.ShapeDtypeStruct((B, 2, NEURONS), jnp.float32),
        scratch_shapes=[
            pltpu.VMEM((N_R, D_SPACE), jnp.float32),
            pltpu.VMEM((NEURONS * N_R, D_SPACE), jnp.float32),
        ],
        compiler_params=pltpu.CompilerParams(
            dimension_semantics=("arbitrary",),
        ),
    )(x, wcat, ce_r, ne_t)


def kernel(x, neuron_emb, W_proj, W_neuron, circuit_emb, top_k):
    del top_k
    wcat = jnp.concatenate([W_proj, W_neuron], axis=0)
    ce_r = circuit_emb[:N_R]
    ne_t = jnp.transpose(neuron_emb, (1, 0, 2)).reshape(NEURONS * N_R, D_SPACE)
    return _run(x, wcat, ce_r, ne_t)


# speedup vs baseline: 1.5019x; 1.0414x over previous
import functools

import jax
import jax.numpy as jnp
from jax import lax
from jax.experimental import pallas as pl
from jax.experimental.pallas import tpu as pltpu

B = 4096
D_MODEL = 2048
D_SPACE = 64
N_R = 64
NEURONS = 64
BT = 512


def _softmax_last(v):
    m = jnp.max(v, axis=-1, keepdims=True)
    e = jnp.exp(v - m)
    return e / jnp.sum(e, axis=-1, keepdims=True)


def _body(x_ref, wp_ref, wn_ref, ce_ref, ne_ref, out_ref, weff_ref, wnb_ref, nn_ref):
    i = pl.program_id(0)

    @pl.when(i == 0)
    def _init():
        ce = ce_ref[...]
        en = ce / (jnp.sqrt(jnp.sum(ce * ce, axis=-1, keepdims=True)) + 1e-12)
        weff_ref[...] = lax.dot_general(en, wp_ref[...], (((1,), (0,)), ((), ())),
                                        preferred_element_type=jnp.float32)
        wnb_ref[...] = wn_ref[...].astype(jnp.bfloat16)
        ne = ne_ref[...]
        nn = ne / (jnp.sqrt(jnp.sum(ne * ne, axis=-1, keepdims=True)) + 1e-12)
        nn_ref[...] = nn.astype(jnp.bfloat16)

    x = x_ref[...]
    x_bf = x.astype(jnp.bfloat16)

    logits = lax.dot_general(x, weff_ref[...], (((1,), (1,)), ((), ())),
                             preferred_element_type=jnp.float32)

    hn = lax.dot_general(x_bf, wnb_ref[...], (((1,), (1,)), ((), ())),
                         preferred_element_type=jnp.float32)

    iota = lax.broadcasted_iota(jnp.int32, logits.shape, 1)
    v1 = jnp.max(logits, axis=-1, keepdims=True)
    i1 = jnp.min(jnp.where(logits == v1, iota, N_R), axis=-1, keepdims=True)
    one1 = iota == i1
    masked = jnp.where(one1, -jnp.inf, logits)
    v2 = jnp.max(masked, axis=-1, keepdims=True)
    i2 = jnp.min(jnp.where(masked == v2, iota, N_R), axis=-1, keepdims=True)
    one2 = iota == i2

    e = jnp.exp(v2 - v1)
    g1 = 1.0 / (1.0 + e)
    g2 = e / (1.0 + e)

    full = lax.dot_general(hn.astype(jnp.bfloat16), nn_ref[...],
                           (((1,), (1,)), ((), ())),
                           preferred_element_type=jnp.float32)
    full3 = full.reshape(BT, NEURONS, N_R)

    nl1 = jnp.sum(jnp.where(one1[:, None, :], full3, 0.0), axis=-1)
    nl2 = jnp.sum(jnp.where(one2[:, None, :], full3, 0.0), axis=-1)

    out_ref[:, 0, :] = _softmax_last(nl1) * g1
    out_ref[:, 1, :] = _softmax_last(nl2) * g2


@functools.partial(jax.jit, static_argnames=())
def _run(x, wp, wn, ce_r, ne_t):
    grid = (B // BT,)
    return pl.pallas_call(
        _body,
        grid=grid,
        in_specs=[
            pl.BlockSpec((BT, D_MODEL), lambda i: (i, 0)),
            pl.BlockSpec((D_SPACE, D_MODEL), lambda i: (0, 0)),
            pl.BlockSpec((D_SPACE, D_MODEL), lambda i: (0, 0)),
            pl.BlockSpec((N_R, D_SPACE), lambda i: (0, 0)),
            pl.BlockSpec((NEURONS * N_R, D_SPACE), lambda i: (0, 0)),
        ],
        out_specs=pl.BlockSpec((BT, 2, NEURONS), lambda i: (i, 0, 0)),
        out_shape=jax.ShapeDtypeStruct((B, 2, NEURONS), jnp.float32),
        scratch_shapes=[
            pltpu.VMEM((N_R, D_MODEL), jnp.float32),
            pltpu.VMEM((D_SPACE, D_MODEL), jnp.bfloat16),
            pltpu.VMEM((NEURONS * N_R, D_SPACE), jnp.bfloat16),
        ],
        compiler_params=pltpu.CompilerParams(
            dimension_semantics=("arbitrary",),
        ),
    )(x, wp, wn, ce_r, ne_t)


def kernel(x, neuron_emb, W_proj, W_neuron, circuit_emb, top_k):
    del top_k
    ce_r = circuit_emb[:N_R]
    ne_t = jnp.transpose(neuron_emb, (1, 0, 2)).reshape(NEURONS * N_R, D_SPACE)
    return _run(x, W_proj, W_neuron, ce_r, ne_t)
